# baseline (speedup 1.0000x reference)
import functools

import jax
import jax.numpy as jnp
from jax import lax
from jax.experimental import pallas as pl
from jax.experimental.pallas import tpu as pltpu
from jax.experimental.pallas import tpu_sc as plsc

NC = 2
NS = 16
NW = NC * NS
IDX_W = 128
FIRE = 5
CB = 16384


RND = 0x8000
MSK = -65536


def _bf16_pack(p, d):
    pi = lax.bitcast_convert_type(p, jnp.int32)
    rnd, msk = jnp.int32(RND), jnp.int32(MSK)
    lo = pi[:, 0:d // 2]
    hi = pi[:, d // 2:d]
    return ((hi + rnd) & msk) | lax.shift_right_logical(lo + rnd, 16)


def _pre_body(d, tt_ref, w_ref, out_ref):
    cb4 = CB // 4
    quarters = []
    for k in range(4):
        pk = lax.dot_general(tt_ref[:, k * cb4:(k + 1) * cb4], w_ref[...],
                             (((0,), (1,)), ((), ())),
                             preferred_element_type=jnp.float32)
        quarters.append(_bf16_pack(pk, d))
    out_ref[...] = jnp.concatenate(quarters, axis=1)


def _pre_project(tableT, W, nblk, d):
    return pl.pallas_call(
        functools.partial(_pre_body, d),
        grid=(nblk,),
        in_specs=[
            pl.BlockSpec((d, CB), lambda j: (0, j)),
            pl.BlockSpec((d, d), lambda j: (0, 0)),
        ],
        out_specs=pl.BlockSpec((CB // 4, 2 * d), lambda j: (j, 0)),
        out_shape=jax.ShapeDtypeStruct((nblk * CB // 4, 2 * d), jnp.int32),
    )(tableT, W)


def _sc_gather(ids3d, pt, n_rows, row_w):
    rows_per_w = n_rows // NW
    idx_rows = rows_per_w // IDX_W
    n_batches = idx_rows // FIRE
    buf_rows = FIRE * IDX_W

    mesh = plsc.VectorSubcoreMesh(core_axis_name="c", subcore_axis_name="s")

    @functools.partial(
        pl.kernel,
        mesh=mesh,
        compiler_params=pltpu.CompilerParams(use_tc_tiling_on_sc=False),
        out_type=jax.ShapeDtypeStruct((n_rows, row_w), jnp.int32),
        scratch_types=[
            pltpu.VMEM((idx_rows, IDX_W), jnp.int32),
            pltpu.VMEM((buf_rows, row_w), jnp.int32),
            pltpu.SemaphoreType.DMA,
        ],
    )
    def gather_kernel(ids_hbm, pt_hbm, out_hbm, idx_v, rows_v, sem):
        wid = lax.axis_index("s") * NC + lax.axis_index("c")
        pltpu.sync_copy(ids_hbm.at[wid], idx_v)
        for c in range(n_batches):
            handles = []
            for j in range(FIRE):
                handles.append(pltpu.async_copy(
                    pt_hbm.at[idx_v.at[c * FIRE + j]],
                    rows_v.at[pl.ds(j * IDX_W, IDX_W)],
                    sem,
                ))
            for h in handles:
                h.wait()
            base = wid * rows_per_w + c * buf_rows
            pltpu.sync_copy(rows_v, out_hbm.at[pl.ds(base, buf_rows)])

    return gather_kernel(ids3d, pt)


def _tc_body(na, bb, lb, d, xt_ref, alignt_ref, embs_ref, q_ref, gate_ref, out_ref):
    rnd, msk = jnp.int32(RND), jnp.int32(MSK)
    g2 = embs_ref[...].reshape(na * bb, 2 * d)
    r0 = lax.broadcasted_iota(jnp.int32, (d, d), 0)
    r1 = lax.broadcasted_iota(jnp.int32, (d, d), 1)
    eye = (r0 == r1).astype(jnp.float32)
    qv = q_ref[0]
    proj_all = None
    for k in range(4):
        wk = g2[:, k * (d // 2):(k + 1) * (d // 2)]
        f_lo = lax.bitcast_convert_type(lax.shift_left(wk, 16), jnp.float32)
        f_hi = lax.bitcast_convert_type(wk & msk, jnp.float32)
        ek = jnp.concatenate([f_lo, f_hi], axis=1)
        pk = lax.dot_general(eye, ek, (((1,), (1,)), ((), ())),
                             preferred_element_type=jnp.float32)
        proj_all = pk if proj_all is None else jnp.where(qv == k, pk, proj_all)
    pi = lax.bitcast_convert_type(proj_all, jnp.int32)
    packed = []
    for a2 in range(na // 2):
        lo = pi[:, (2 * a2) * bb:(2 * a2 + 1) * bb]
        hi = pi[:, (2 * a2 + 1) * bb:(2 * a2 + 2) * bb]
        packed.append(((hi + rnd) & msk) | lax.shift_right_logical(lo + rnd, 16))
    t = jnp.tanh(gate_ref[0, 0])
    lch = 4
    for lc in range(0, lb, lch):
        al = alignt_ref[pl.ds(lc, lch), :]
        al2 = al >> 1
        shamt = ((al & 1) ^ 1) << 4
        acc = jnp.broadcast_to(packed[0][None, :, :], (lch, d, bb))
        for a2 in range(1, na // 2):
            acc = jnp.where((al2 == a2)[:, None, :], packed[a2][None, :, :], acc)
        tmp = lax.shift_left(acc, jnp.broadcast_to(shamt[:, None, :], acc.shape))
        sel = lax.bitcast_convert_type(tmp & msk, jnp.float32)
        out_ref[pl.ds(lc, lch)] = xt_ref[pl.ds(lc, lch)] + t * sel


def kernel(x, anchor_ids, anchor_alignment, table, W, gate):
    b, ll, d = x.shape
    na = anchor_ids.shape[1]
    v = table.shape[0]
    bb = 128
    lb = 40

    nblk = (v + CB - 1) // CB
    cb4 = CB // 4

    tableT = table.T
    pt = _pre_project(tableT, W, nblk, d)

    ids_p = anchor_ids.astype(jnp.int32).T.reshape(-1)
    ib, off = ids_p // CB, ids_p % CB
    ids2 = ib * cb4 + off % cb4
    ids3d = ids2.reshape(NW, b * na // (NW * IDX_W), IDX_W)
    embs = _sc_gather(ids3d, pt, b * na, 2 * d)
    embs3 = embs.reshape(na, b, 2 * d)
    qm = (off // cb4).reshape(na, b // bb, bb).transpose(1, 0, 2) \
        .reshape(b // bb, 1, na * bb)

    xt = jnp.transpose(x, (1, 2, 0))
    alignt = anchor_alignment.astype(jnp.int32).T
    gate2 = gate.reshape(1, 1)

    outt = pl.pallas_call(
        functools.partial(_tc_body, na, bb, lb, d),
        grid=(b // bb, ll // lb),
        in_specs=[
            pl.BlockSpec((lb, d, bb), lambda i, j: (j, 0, i)),
            pl.BlockSpec((lb, bb), lambda i, j: (j, i)),
            pl.BlockSpec((na, bb, 2 * d), lambda i, j: (0, i, 0)),
            pl.BlockSpec((1, 1, na * bb), lambda i, j: (i, 0, 0)),
            pl.BlockSpec((1, 1), lambda i, j: (0, 0)),
        ],
        out_specs=pl.BlockSpec((lb, d, bb), lambda i, j: (j, 0, i)),
        out_shape=jax.ShapeDtypeStruct((ll, d, b), jnp.float32),
    )(xt, alignt, embs3, qm, gate2)

    return jnp.transpose(outt, (2, 0, 1))


# speedup vs baseline: 1.5288x; 1.5288x over previous
import functools

import jax
import jax.numpy as jnp
from jax import lax
from jax.experimental import pallas as pl
from jax.experimental.pallas import tpu as pltpu
from jax.experimental.pallas import tpu_sc as plsc

NC = 2
NS = 16
NW = NC * NS
IDX_W = 128
FIRE = 5
CB = 16384


def _pre_body(d, tt_ref, w_ref, out_ref):
    cb2 = CB // 2
    pa = lax.dot_general(tt_ref[:, 0:cb2], w_ref[...], (((0,), (1,)), ((), ())),
                         preferred_element_type=jnp.float32)
    pb = lax.dot_general(tt_ref[:, cb2:CB], w_ref[...], (((0,), (1,)), ((), ())),
                         preferred_element_type=jnp.float32)
    out_ref[...] = jnp.concatenate([pa, pb], axis=1)


def _pre_project(tableT, W, nblk, d):
    return pl.pallas_call(
        functools.partial(_pre_body, d),
        grid=(nblk,),
        in_specs=[
            pl.BlockSpec((d, CB), lambda j: (0, j)),
            pl.BlockSpec((d, d), lambda j: (0, 0)),
        ],
        out_specs=pl.BlockSpec((CB // 2, 2 * d), lambda j: (j, 0)),
        out_shape=jax.ShapeDtypeStruct((nblk * CB // 2, 2 * d), jnp.float32),
    )(tableT, W)


def _sc_gather(ids3d, pt, n_rows, row_w):
    rows_per_w = n_rows // NW
    idx_rows = rows_per_w // IDX_W
    n_batches = idx_rows // FIRE
    buf_rows = FIRE * IDX_W

    mesh = plsc.VectorSubcoreMesh(core_axis_name="c", subcore_axis_name="s")

    @functools.partial(
        pl.kernel,
        mesh=mesh,
        compiler_params=pltpu.CompilerParams(use_tc_tiling_on_sc=False),
        out_type=jax.ShapeDtypeStruct((n_rows, 2 * row_w), jnp.float32),
        scratch_types=[
            pltpu.VMEM((idx_rows, IDX_W), jnp.int32),
            pltpu.VMEM((buf_rows, row_w), jnp.float32),
            pltpu.SemaphoreType.DMA,
        ],
    )
    def gather_kernel(ids_hbm, pt_hbm, out_hbm, idx_v, rows_v, sem):
        wid = lax.axis_index("s") * NC + lax.axis_index("c")
        pltpu.sync_copy(ids_hbm.at[wid], idx_v)
        for c in range(n_batches):
            handles = []
            for j in range(FIRE):
                handles.append(pltpu.async_copy(
                    pt_hbm.at[idx_v.at[c * FIRE + j]],
                    rows_v.at[pl.ds(j * IDX_W, IDX_W)],
                    sem,
                ))
            for h in handles:
                h.wait()
            base = wid * rows_per_w + c * buf_rows
            pltpu.sync_copy(
                rows_v, out_hbm.at[pl.ds(base, buf_rows), pl.ds(0, row_w)])

    return gather_kernel(ids3d, pt)


def _tc_body(na, bb, lb, d, xt_ref, alignt_ref, embs_ref, gate_ref, out_ref):
    embs2 = embs_ref[...][:, :, 0:d].reshape(na * bb, d)
    r0 = lax.broadcasted_iota(jnp.int32, (d, d), 0)
    r1 = lax.broadcasted_iota(jnp.int32, (d, d), 1)
    eye = (r0 == r1).astype(jnp.float32)
    proj_all = lax.dot_general(eye, embs2, (((1,), (1,)), ((), ())),
                               preferred_element_type=jnp.float32)
    pi = lax.bitcast_convert_type(proj_all, jnp.int32)
    rnd = jnp.int32(0x8000)
    msk = jnp.int32(-65536)
    packed = []
    for a2 in range(na // 2):
        lo = pi[:, (2 * a2) * bb:(2 * a2 + 1) * bb]
        hi = pi[:, (2 * a2 + 1) * bb:(2 * a2 + 2) * bb]
        packed.append(((hi + rnd) & msk) | lax.shift_right_logical(lo + rnd, 16))
    t = jnp.tanh(gate_ref[0, 0])
    lch = 4
    for lc in range(0, lb, lch):
        al = alignt_ref[pl.ds(lc, lch), :]
        al2 = al >> 1
        shamt = ((al & 1) ^ 1) << 4
        acc = jnp.broadcast_to(packed[0][None, :, :], (lch, d, bb))
        for a2 in range(1, na // 2):
            acc = jnp.where((al2 == a2)[:, None, :], packed[a2][None, :, :], acc)
        tmp = lax.shift_left(acc, jnp.broadcast_to(shamt[:, None, :], acc.shape))
        sel = lax.bitcast_convert_type(tmp & msk, jnp.float32)
        out_ref[pl.ds(lc, lch)] = xt_ref[pl.ds(lc, lch)] + t * sel


def kernel(x, anchor_ids, anchor_alignment, table, W, gate):
    b, ll, d = x.shape
    na = anchor_ids.shape[1]
    v = table.shape[0]
    bb = 128
    lb = 40

    nblk = (v + CB - 1) // CB
    cb2 = CB // 2

    tableT = table.T
    pt = _pre_project(tableT, W, nblk, d)

    pt2 = pt.reshape(nblk * CB, d)
    ids_p = anchor_ids.astype(jnp.int32).T.reshape(-1)
    ib, off = ids_p // CB, ids_p % CB
    ids2 = 2 * (ib * cb2 + off % cb2) + off // cb2
    ids3d = ids2.reshape(NW, b * na // (NW * IDX_W), IDX_W)
    embs = _sc_gather(ids3d, pt2, b * na, d)
    embs3 = embs.reshape(na, b, 2 * d)

    xt = jnp.transpose(x, (1, 2, 0))
    alignt = anchor_alignment.astype(jnp.int32).T
    gate2 = gate.reshape(1, 1)

    outt = pl.pallas_call(
        functools.partial(_tc_body, na, bb, lb, d),
        grid=(b // bb, ll // lb),
        in_specs=[
            pl.BlockSpec((lb, d, bb), lambda i, j: (j, 0, i)),
            pl.BlockSpec((lb, bb), lambda i, j: (j, i)),
            pl.BlockSpec((na, bb, 2 * d), lambda i, j: (0, i, 0)),
            pl.BlockSpec((1, 1), lambda i, j: (0, 0)),
        ],
        out_specs=pl.BlockSpec((lb, d, bb), lambda i, j: (j, 0, i)),
        out_shape=jax.ShapeDtypeStruct((ll, d, b), jnp.float32),
    )(xt, alignt, embs3, gate2)

    return jnp.transpose(outt, (2, 0, 1))
